# Initial kernel scaffold; baseline (speedup 1.0000x reference)
import jax
import jax.numpy as jnp
from jax.experimental import pallas as pl


def kernel(y_pred):
    raise NotImplementedError("write your pallas kernel here")


# speedup vs baseline: 7.2475x; 7.2475x over previous
import functools

import jax
import jax.numpy as jnp
import numpy as np
from jax import lax
from jax.experimental import pallas as pl
from jax.experimental.pallas import tpu as pltpu
from jax.experimental.pallas import tpu_sc as plsc

B = 64
S = 8
N = 32768
K = 50
OUTW = 64
NC = 2
NS = 16
NW = NC * NS
ROWS_PER_W = B // NW

_MININT = np.int32(-(2 ** 31))


def _lane():
    return lax.iota(jnp.int32, 16)


def _key_of(b):
    return b ^ jnp.where(b >= 0, jnp.int32(0), jnp.int32(0x7FFFFFFF))


def _digit_of(k, idx, r):
    ku = k ^ _MININT
    if r == 0:
        return lax.shift_right_logical(ku, 24)
    if r == 1:
        return lax.shift_right_logical(ku, 16) & 0xFF
    if r == 2:
        return lax.shift_right_logical(ku, 8) & 0xFF
    if r == 3:
        return ku & 0xFF
    if r == 4:
        return lax.shift_right_logical(idx, 7)
    return idx & 0x7F


def _topk_body(y_hbm, out_hbm, row_v, au_v, ai_v, hist_v, tot_v, s_v,
               su_v, si_v, ob_v):
    lane = _lane()
    zero16 = jnp.zeros((16,), jnp.int32)
    one16 = jnp.ones((16,), jnp.int32)
    neg16 = jnp.full((16,), -1, jnp.int32)
    lanebase = lane * 256

    wid = lax.axis_index("s") * NC + lax.axis_index("c")

    def load_elems(r, base, nact):
        if r == 0:
            k = _key_of(row_v[pl.ds(base, 16)])
            idx = base + lane
        else:
            k = au_v[pl.ds(base, 16)]
            idx = ai_v[pl.ds(base, 16)]
        valid = (base + lane) < nact
        return k, idx, valid

    def run_round(r, k_rem, nsel, nact):
        nvec = lax.div(nact + 15, jnp.int32(16))

        def z_body(i, c):
            hist_v[pl.ds(i * 16, 16)] = zero16
            return c
        lax.fori_loop(0, 256, z_body, 0)

        def h_body(i, c):
            k, idx, valid = load_elems(r, i * 16, nact)
            d = _digit_of(k, idx, r)
            plsc.addupdate_scatter(hist_v, [lanebase + d], one16, mask=valid)
            return c
        lax.fori_loop(0, nvec, h_body, 0)

        def c_body(j, c):
            acc = zero16
            for l in range(16):
                acc = acc + hist_v[pl.ds(l * 256 + j * 16, 16)]
            tot_v[pl.ds(j * 16, 16)] = acc
            return c
        lax.fori_loop(0, 16, c_body, 0)

        def s_body(jj, carry):
            c_hi, bstar = carry
            j = 15 - jj
            t = tot_v[pl.ds(j * 16, 16)]
            cs = plsc.cumsum(lax.rev(t, (0,)))
            s_vec = lax.rev(cs, (0,)) + c_hi
            s_v[pl.ds(j * 16, 16)] = s_vec
            bins = j * 16 + lane
            cand = jnp.where(s_vec >= k_rem, bins, -1)
            return c_hi + jnp.max(cs), jnp.maximum(bstar, jnp.max(cand))
        _, bstar = lax.fori_loop(0, 16, s_body, (jnp.int32(0), jnp.int32(-1)))

        bsplat = jnp.full((16,), bstar, jnp.int32)
        count_eq = jnp.max(plsc.load_gather(tot_v, [bsplat]))
        cum_before = jnp.max(plsc.load_gather(s_v, [bsplat])) - count_eq
        k_rem2 = k_rem - cum_before
        all_eq = k_rem2 == count_eq

        def p_body(i, carry):
            ns, na = carry
            k, idx, valid = load_elems(r, i * 16, nact)
            d = _digit_of(k, idx, r)
            gt = (d > bstar) & valid
            eq = (d == bstar) & valid
            m_sel = gt | (eq & all_eq)
            m_keep = eq & jnp.logical_not(all_eq)
            pc_s = plsc.cumsum(jnp.where(m_sel, 1, 0))
            plsc.store_scatter(su_v, [ns + pc_s - 1], k, mask=m_sel)
            plsc.store_scatter(si_v, [ns + pc_s - 1], idx, mask=m_sel)
            pc_k = plsc.cumsum(jnp.where(m_keep, 1, 0))
            plsc.store_scatter(au_v, [na + pc_k - 1], k, mask=m_keep)
            plsc.store_scatter(ai_v, [na + pc_k - 1], idx, mask=m_keep)
            return ns + jnp.max(pc_s), na + jnp.max(pc_k)
        nsel, nact_new = lax.fori_loop(0, nvec, p_body, (nsel, jnp.int32(0)))

        k_rem = jnp.where(all_eq, jnp.int32(0), k_rem2)
        return k_rem, nsel, nact_new

    for rr in range(ROWS_PER_W):
        out_row = wid * ROWS_PER_W + rr
        in_row = (B - 1) - out_row

        pltpu.sync_copy(y_hbm.at[in_row], row_v)

        for q in range(4):
            su_v[pl.ds(q * 16, 16)] = jnp.full((16,), _MININT, jnp.int32)
            si_v[pl.ds(q * 16, 16)] = neg16
        ob_v[pl.ds(48, 16)] = neg16

        k_rem = jnp.int32(K)
        nsel = jnp.int32(0)
        nact = jnp.int32(N)
        for r in range(6):
            k_rem, nsel, nact = run_round(r, k_rem, nsel, nact)

        def srt_body(j, carry):
            u0, u1, u2, u3, i0, i1, i2, i3 = carry
            s = jnp.max(jnp.maximum(jnp.maximum(u0, u1), jnp.maximum(u2, u3)))
            c0 = jnp.where(u0 == s, i0, -1)
            c1 = jnp.where(u1 == s, i1, -1)
            c2 = jnp.where(u2 == s, i2, -1)
            c3 = jnp.where(u3 == s, i3, -1)
            mi = jnp.max(jnp.maximum(jnp.maximum(c0, c1), jnp.maximum(c2, c3)))
            plsc.store_scatter(ob_v, [jnp.full((16,), j, jnp.int32)],
                               jnp.full((16,), mi, jnp.int32), mask=lane == 0)
            outs = []
            for (u, iv) in ((u0, i0), (u1, i1), (u2, i2), (u3, i3)):
                hit = (u == s) & (iv == mi)
                outs.append(jnp.where(hit, _MININT, u))
                outs.append(jnp.where(hit, -1, iv))
            return outs[0], outs[2], outs[4], outs[6], outs[1], outs[3], outs[5], outs[7]

        init = (su_v[pl.ds(0, 16)], su_v[pl.ds(16, 16)],
                su_v[pl.ds(32, 16)], su_v[pl.ds(48, 16)],
                si_v[pl.ds(0, 16)], si_v[pl.ds(16, 16)],
                si_v[pl.ds(32, 16)], si_v[pl.ds(48, 16)])
        lax.fori_loop(0, K, srt_body, init)

        pltpu.sync_copy(ob_v, out_hbm.at[out_row])


@functools.cache
def _build_topk_sc():
    return pl.kernel(
        _topk_body,
        out_type=jax.ShapeDtypeStruct((B, OUTW), jnp.int32),
        mesh=plsc.VectorSubcoreMesh(core_axis_name="c", subcore_axis_name="s",
                                    num_cores=NC, num_subcores=NS),
        scratch_types=[
            pltpu.VMEM((N,), jnp.int32),
            pltpu.VMEM((N,), jnp.int32),
            pltpu.VMEM((N,), jnp.int32),
            pltpu.VMEM((16 * 256,), jnp.int32),
            pltpu.VMEM((256,), jnp.int32),
            pltpu.VMEM((256,), jnp.int32),
            pltpu.VMEM((OUTW,), jnp.int32),
            pltpu.VMEM((OUTW,), jnp.int32),
            pltpu.VMEM((OUTW,), jnp.int32),
        ],
        compiler_params=pltpu.CompilerParams(needs_layout_passes=False),
    )


def kernel(y_pred):
    bits = jax.lax.bitcast_convert_type(y_pred[:, -1], jnp.int32)
    return _build_topk_sc()(bits)[:, :K]


# speedup vs baseline: 7.6723x; 1.0586x over previous
import functools

import jax
import jax.numpy as jnp
import numpy as np
from jax import lax
from jax.experimental import pallas as pl
from jax.experimental.pallas import tpu as pltpu
from jax.experimental.pallas import tpu_sc as plsc

B = 64
S = 8
N = 32768
K = 50
OUTW = 64
NC = 2
NS = 16
NW = NC * NS
ROWS_PER_W = B // NW

_MININT = np.int32(-(2 ** 31))


def _lane():
    return lax.iota(jnp.int32, 16)


def _key_of(b):
    return b ^ jnp.where(b >= 0, jnp.int32(0), jnp.int32(0x7FFFFFFF))


def _digit_of(k, idx, r):
    ku = k ^ _MININT
    if r == 0:
        return lax.shift_right_logical(ku, 24)
    if r == 1:
        return lax.shift_right_logical(ku, 16) & 0xFF
    if r == 2:
        return lax.shift_right_logical(ku, 8) & 0xFF
    if r == 3:
        return ku & 0xFF
    if r == 4:
        return lax.shift_right_logical(idx, 7)
    return idx & 0x7F


def _topk_body(y_hbm, out_hbm, row_v, au_v, ai_v, hist_v, tot_v, s_v,
               su_v, si_v, ob_v):
    lane = _lane()
    zero16 = jnp.zeros((16,), jnp.int32)
    one16 = jnp.ones((16,), jnp.int32)
    neg16 = jnp.full((16,), -1, jnp.int32)
    lanebase = lane * 256

    wid = lax.axis_index("s") * NC + lax.axis_index("c")

    def load_elems(r, base, nact):
        if r == 0:
            k = _key_of(row_v[pl.ds(base, 16)])
            idx = base + lane
        else:
            k = au_v[pl.ds(base, 16)]
            idx = ai_v[pl.ds(base, 16)]
        valid = (base + lane) < nact
        return k, idx, valid

    def run_round(r, k_rem, ns_vec, nact):
        nvec = lax.div(nact + 15, jnp.int32(16))
        live = nact > 0
        n16 = jnp.where(live, jnp.int32(16), jnp.int32(0))
        n256 = jnp.where(live, jnp.int32(256), jnp.int32(0))

        def z_body(i, c):
            hist_v[pl.ds(i * 16, 16)] = zero16
            return c
        lax.fori_loop(0, n256, z_body, 0)

        def h_body(i, c):
            k, idx, valid = load_elems(r, i * 16, nact)
            d = _digit_of(k, idx, r)
            plsc.addupdate_scatter(hist_v, [lanebase + d], one16, mask=valid)
            return c
        lax.fori_loop(0, nvec, h_body, 0)

        def c_body(j, c):
            acc = zero16
            for l in range(16):
                acc = acc + hist_v[pl.ds(l * 256 + j * 16, 16)]
            tot_v[pl.ds(j * 16, 16)] = acc
            return c
        lax.fori_loop(0, n16, c_body, 0)

        def s_body(jj, carry):
            c_hi, bstar = carry
            j = 15 - jj
            t = tot_v[pl.ds(j * 16, 16)]
            cs = plsc.cumsum(lax.rev(t, (0,)))
            s_vec = lax.rev(cs, (0,)) + c_hi
            s_v[pl.ds(j * 16, 16)] = s_vec
            bins = j * 16 + lane
            cand = jnp.where(s_vec >= k_rem, bins, -1)
            return c_hi + jnp.max(cs), jnp.maximum(bstar, jnp.max(cand))
        _, bstar = lax.fori_loop(0, n16, s_body, (jnp.int32(0), jnp.int32(-1)))

        bsplat = jnp.full((16,), bstar, jnp.int32)
        count_eq = jnp.max(plsc.load_gather(tot_v, [bsplat]))
        cum_before = jnp.max(plsc.load_gather(s_v, [bsplat])) - count_eq
        k_rem2 = k_rem - cum_before
        all_eq = k_rem2 == count_eq

        if r == 0:
            t_lo = jnp.full((16,), (bstar << 24) ^ jnp.int32(_MININT), jnp.int32)
            t_hi = jnp.full((16,), ((bstar + 1) << 24) ^ jnp.int32(_MININT),
                            jnp.int32)
            not_top = bstar < 255

        def p_body(i, carry):
            ns, na = carry
            k, idx, valid = load_elems(r, i * 16, nact)
            if r == 0:
                gt = (k >= t_hi) & not_top
                eq = (k >= t_lo) & jnp.logical_not(gt)
            else:
                d = _digit_of(k, idx, r)
                gt = (d > bstar) & valid
                eq = (d == bstar) & valid
            m_sel = gt | (eq & all_eq)
            m_keep = eq & jnp.logical_not(all_eq)
            pc_s = plsc.cumsum(m_sel.astype(jnp.int32))
            plsc.store_scatter(su_v, [ns + pc_s - 1], k, mask=m_sel)
            plsc.store_scatter(si_v, [ns + pc_s - 1], idx, mask=m_sel)
            pc_k = plsc.cumsum(m_keep.astype(jnp.int32))
            plsc.store_scatter(au_v, [na + pc_k - 1], k, mask=m_keep)
            plsc.store_scatter(ai_v, [na + pc_k - 1], idx, mask=m_keep)
            ns = ns + plsc.all_reduce_population_count(m_sel)
            na = na + plsc.all_reduce_population_count(m_keep)
            return ns, na
        ns_vec, na_vec = lax.fori_loop(0, nvec, p_body, (ns_vec, zero16))

        k_rem = jnp.where(all_eq, jnp.int32(0), k_rem2)
        nact_new = jnp.where(live, jnp.max(na_vec), jnp.int32(0))
        return k_rem, ns_vec, nact_new

    for rr in range(ROWS_PER_W):
        out_row = wid * ROWS_PER_W + rr
        in_row = (B - 1) - out_row

        pltpu.sync_copy(y_hbm.at[in_row], row_v)

        for q in range(4):
            su_v[pl.ds(q * 16, 16)] = jnp.full((16,), _MININT, jnp.int32)
            si_v[pl.ds(q * 16, 16)] = neg16
        ob_v[pl.ds(48, 16)] = neg16

        k_rem = jnp.int32(K)
        ns_vec = zero16
        nact = jnp.int32(N)
        for r in range(6):
            k_rem, ns_vec, nact = run_round(r, k_rem, ns_vec, nact)

        def srt_body(j, carry):
            u0, u1, u2, u3, i0, i1, i2, i3 = carry
            s = jnp.max(jnp.maximum(jnp.maximum(u0, u1), jnp.maximum(u2, u3)))
            c0 = jnp.where(u0 == s, i0, -1)
            c1 = jnp.where(u1 == s, i1, -1)
            c2 = jnp.where(u2 == s, i2, -1)
            c3 = jnp.where(u3 == s, i3, -1)
            mi = jnp.max(jnp.maximum(jnp.maximum(c0, c1), jnp.maximum(c2, c3)))
            plsc.store_scatter(ob_v, [jnp.full((16,), j, jnp.int32)],
                               jnp.full((16,), mi, jnp.int32), mask=lane == 0)
            outs = []
            for (u, iv) in ((u0, i0), (u1, i1), (u2, i2), (u3, i3)):
                hit = (u == s) & (iv == mi)
                outs.append(jnp.where(hit, _MININT, u))
                outs.append(jnp.where(hit, -1, iv))
            return outs[0], outs[2], outs[4], outs[6], outs[1], outs[3], outs[5], outs[7]

        init = (su_v[pl.ds(0, 16)], su_v[pl.ds(16, 16)],
                su_v[pl.ds(32, 16)], su_v[pl.ds(48, 16)],
                si_v[pl.ds(0, 16)], si_v[pl.ds(16, 16)],
                si_v[pl.ds(32, 16)], si_v[pl.ds(48, 16)])
        lax.fori_loop(0, K, srt_body, init)

        pltpu.sync_copy(ob_v, out_hbm.at[out_row])


@functools.cache
def _build_topk_sc():
    return pl.kernel(
        _topk_body,
        out_type=jax.ShapeDtypeStruct((B, OUTW), jnp.int32),
        mesh=plsc.VectorSubcoreMesh(core_axis_name="c", subcore_axis_name="s",
                                    num_cores=NC, num_subcores=NS),
        scratch_types=[
            pltpu.VMEM((N,), jnp.int32),
            pltpu.VMEM((N,), jnp.int32),
            pltpu.VMEM((N,), jnp.int32),
            pltpu.VMEM((16 * 256,), jnp.int32),
            pltpu.VMEM((256,), jnp.int32),
            pltpu.VMEM((256,), jnp.int32),
            pltpu.VMEM((OUTW,), jnp.int32),
            pltpu.VMEM((OUTW,), jnp.int32),
            pltpu.VMEM((OUTW,), jnp.int32),
        ],
        compiler_params=pltpu.CompilerParams(needs_layout_passes=False),
    )


def kernel(y_pred):
    bits = jax.lax.bitcast_convert_type(y_pred[:, -1], jnp.int32)
    return _build_topk_sc()(bits)[:, :K]


# speedup vs baseline: 7.9652x; 1.0382x over previous
import functools

import jax
import jax.numpy as jnp
import numpy as np
from jax import lax
from jax.experimental import pallas as pl
from jax.experimental.pallas import tpu as pltpu
from jax.experimental.pallas import tpu_sc as plsc

B = 64
S = 8
N = 32768
K = 50
OUTW = 64
NC = 2
NS = 16
NW = NC * NS
ROWS_PER_W = B // NW
UNROLL = 4

_MININT = np.int32(-(2 ** 31))


def _lane():
    return lax.iota(jnp.int32, 16)


def _bmax(x):
    for d in (1, 2, 4, 8):
        x = jnp.maximum(x, jnp.take(x, _lane() ^ d))
    return x


def _key_of(b):
    return b ^ jnp.where(b >= 0, jnp.int32(0), jnp.int32(0x7FFFFFFF))


def _digit_of(k, idx, r):
    ku = k ^ _MININT
    if r == 0:
        return lax.shift_right_logical(ku, 24)
    if r == 1:
        return lax.shift_right_logical(ku, 16) & 0xFF
    if r == 2:
        return lax.shift_right_logical(ku, 8) & 0xFF
    if r == 3:
        return ku & 0xFF
    if r == 4:
        return lax.shift_right_logical(idx, 7)
    return idx & 0x7F


def _topk_body(y_hbm, out_hbm, row_v, au_v, ai_v, hist_v, tot_v, s_v,
               su_v, si_v, ob_v):
    lane = _lane()
    zero16 = jnp.zeros((16,), jnp.int32)
    one16 = jnp.ones((16,), jnp.int32)
    neg16 = jnp.full((16,), -1, jnp.int32)
    min16 = jnp.full((16,), _MININT, jnp.int32)
    lanebase = lane * 256

    wid = lax.axis_index("s") * NC + lax.axis_index("c")

    def load_elems(r, base, nact):
        if r == 0:
            k = _key_of(row_v[pl.ds(base, 16)])
            idx = base + lane
        else:
            k = au_v[pl.ds(base, 16)]
            idx = ai_v[pl.ds(base, 16)]
        valid = (base + lane) < nact
        return k, idx, valid

    def run_round(r, k_rem_v, ns_vec, nact):
        nvec_u = lax.div(nact + 16 * UNROLL - 1, jnp.int32(16 * UNROLL))
        live = nact > 0
        n16 = jnp.where(live, jnp.int32(16), jnp.int32(0))
        n64 = jnp.where(live, jnp.int32(64), jnp.int32(0))

        def z_body(i, c):
            for u in range(4):
                hist_v[pl.ds(i * 64 + u * 16, 16)] = zero16
            return c
        lax.fori_loop(0, n64, z_body, 0)

        def h_body(i, c):
            for u in range(UNROLL):
                k, idx, valid = load_elems(r, i * (16 * UNROLL) + u * 16, nact)
                d = _digit_of(k, idx, r)
                plsc.addupdate_scatter(hist_v, [lanebase + d], one16,
                                       mask=valid)
            return c
        lax.fori_loop(0, nvec_u, h_body, 0)

        def c_body(j, c):
            acc = zero16
            for l in range(16):
                acc = acc + hist_v[pl.ds(l * 256 + j * 16, 16)]
            tot_v[pl.ds(j * 16, 16)] = acc
            return c
        lax.fori_loop(0, n16, c_body, 0)

        def s_body(jj, carry):
            c_hi_v, bstar_v = carry
            j = 15 - jj
            t = tot_v[pl.ds(j * 16, 16)]
            cs = plsc.cumsum(lax.rev(t, (0,)))
            s_vec = lax.rev(cs, (0,)) + c_hi_v
            s_v[pl.ds(j * 16, 16)] = s_vec
            bins = j * 16 + lane
            cand = jnp.where(s_vec >= k_rem_v, bins, -1)
            return c_hi_v + _bmax(cs), jnp.maximum(bstar_v, _bmax(cand))
        _, bstar_v = lax.fori_loop(0, n16, s_body, (zero16, neg16))
        bstar_v = jnp.maximum(bstar_v, zero16)

        count_eq_v = plsc.load_gather(tot_v, [bstar_v])
        cum_before_v = plsc.load_gather(s_v, [bstar_v]) - count_eq_v
        k_rem2_v = k_rem_v - cum_before_v
        all_eq_v = k_rem2_v == count_eq_v

        if r == 0:
            t_lo_v = (bstar_v << 24) ^ min16
            t_hi_v = ((bstar_v + 1) << 24) ^ min16
            not_top_v = bstar_v < 255

        def p_body(i, carry):
            ns, na = carry
            for u in range(UNROLL):
                k, idx, valid = load_elems(r, i * (16 * UNROLL) + u * 16, nact)
                if r == 0:
                    gt = (k >= t_hi_v) & not_top_v
                    eq = (k >= t_lo_v) & jnp.logical_not(gt)
                else:
                    d = _digit_of(k, idx, r)
                    gt = (d > bstar_v) & valid
                    eq = (d == bstar_v) & valid
                m_sel = gt | (eq & all_eq_v)
                m_keep = eq & jnp.logical_not(all_eq_v)
                pc_s = plsc.cumsum(m_sel.astype(jnp.int32))
                plsc.store_scatter(su_v, [ns + pc_s - 1], k, mask=m_sel)
                plsc.store_scatter(si_v, [ns + pc_s - 1], idx, mask=m_sel)
                pc_k = plsc.cumsum(m_keep.astype(jnp.int32))
                plsc.store_scatter(au_v, [na + pc_k - 1], k, mask=m_keep)
                plsc.store_scatter(ai_v, [na + pc_k - 1], idx, mask=m_keep)
                ns = ns + plsc.all_reduce_population_count(m_sel)
                na = na + plsc.all_reduce_population_count(m_keep)
            return ns, na
        ns_vec, na_vec = lax.fori_loop(0, nvec_u, p_body, (ns_vec, zero16))

        k_rem_v = jnp.where(all_eq_v, zero16, k_rem2_v)
        nact_new = jnp.max(na_vec)
        return k_rem_v, ns_vec, nact_new

    for rr in range(ROWS_PER_W):
        out_row = wid * ROWS_PER_W + rr
        in_row = (B - 1) - out_row

        pltpu.sync_copy(y_hbm.at[in_row], row_v)

        for q in range(4):
            su_v[pl.ds(q * 16, 16)] = min16
            si_v[pl.ds(q * 16, 16)] = neg16
        ob_v[pl.ds(48, 16)] = neg16

        k_rem_v = jnp.full((16,), K, jnp.int32)
        ns_vec = zero16
        nact = jnp.int32(N)
        for r in range(6):
            k_rem_v, ns_vec, nact = run_round(r, k_rem_v, ns_vec, nact)

        def srt_body(j, carry):
            u0, u1, u2, u3, i0, i1, i2, i3 = carry
            s = _bmax(jnp.maximum(jnp.maximum(u0, u1), jnp.maximum(u2, u3)))
            c0 = jnp.where(u0 == s, i0, -1)
            c1 = jnp.where(u1 == s, i1, -1)
            c2 = jnp.where(u2 == s, i2, -1)
            c3 = jnp.where(u3 == s, i3, -1)
            mi = _bmax(jnp.maximum(jnp.maximum(c0, c1), jnp.maximum(c2, c3)))
            plsc.store_scatter(ob_v, [jnp.full((16,), j, jnp.int32)], mi)
            outs = []
            for (u, iv) in ((u0, i0), (u1, i1), (u2, i2), (u3, i3)):
                hit = (u == s) & (iv == mi)
                outs.append(jnp.where(hit, min16, u))
                outs.append(jnp.where(hit, neg16, iv))
            return (outs[0], outs[2], outs[4], outs[6],
                    outs[1], outs[3], outs[5], outs[7])

        init = (su_v[pl.ds(0, 16)], su_v[pl.ds(16, 16)],
                su_v[pl.ds(32, 16)], su_v[pl.ds(48, 16)],
                si_v[pl.ds(0, 16)], si_v[pl.ds(16, 16)],
                si_v[pl.ds(32, 16)], si_v[pl.ds(48, 16)])
        lax.fori_loop(0, K, srt_body, init)

        pltpu.sync_copy(ob_v, out_hbm.at[out_row])


@functools.cache
def _build_topk_sc():
    return pl.kernel(
        _topk_body,
        out_type=jax.ShapeDtypeStruct((B, OUTW), jnp.int32),
        mesh=plsc.VectorSubcoreMesh(core_axis_name="c", subcore_axis_name="s",
                                    num_cores=NC, num_subcores=NS),
        scratch_types=[
            pltpu.VMEM((N,), jnp.int32),
            pltpu.VMEM((N,), jnp.int32),
            pltpu.VMEM((N,), jnp.int32),
            pltpu.VMEM((16 * 256,), jnp.int32),
            pltpu.VMEM((256,), jnp.int32),
            pltpu.VMEM((256,), jnp.int32),
            pltpu.VMEM((OUTW,), jnp.int32),
            pltpu.VMEM((OUTW,), jnp.int32),
            pltpu.VMEM((OUTW,), jnp.int32),
        ],
        compiler_params=pltpu.CompilerParams(needs_layout_passes=False),
    )


def kernel(y_pred):
    bits = jax.lax.bitcast_convert_type(y_pred[:, -1], jnp.int32)
    return _build_topk_sc()(bits)[:, :K]


# speedup vs baseline: 8.7197x; 1.0947x over previous
import functools

import jax
import jax.numpy as jnp
import numpy as np
from jax import lax
from jax.experimental import pallas as pl
from jax.experimental.pallas import tpu as pltpu
from jax.experimental.pallas import tpu_sc as plsc

B = 64
N = 32768
K = 50
OUTW = 64
NC = 2
NS = 16
NW = NC * NS
ROWS_PER_W = B // NW
CAP = N // 16

_MININT = np.int32(-(2 ** 31))


def _lane():
    return lax.iota(jnp.int32, 16)


def _bmax(x):
    for d in (1, 2, 4, 8):
        x = jnp.maximum(x, jnp.take(x, _lane() ^ d))
    return x


def _key_of(b):
    return b ^ jnp.where(b >= 0, jnp.int32(0), jnp.int32(0x7FFFFFFF))


def _digit_of(k, idx, r):
    ku = k ^ _MININT
    if r == 0:
        return lax.shift_right_logical(ku, 24)
    if r == 1:
        return lax.shift_right_logical(ku, 16) & 0xFF
    if r == 2:
        return lax.shift_right_logical(ku, 8) & 0xFF
    if r == 3:
        return ku & 0xFF
    if r == 4:
        return lax.shift_right_logical(idx, 7)
    return idx & 0x7F


def _topk_body(y_hbm, out_hbm, row_v, au_v, ai_v, h0, h1, h2, h3,
               tot_v, s_v, su_v, si_v, ob_v):
    lane = _lane()
    hists = (h0, h1, h2, h3)
    zero16 = jnp.zeros((16,), jnp.int32)
    one16 = jnp.ones((16,), jnp.int32)
    neg16 = jnp.full((16,), -1, jnp.int32)
    min16 = jnp.full((16,), _MININT, jnp.int32)
    lanebase = lane * 256
    bbase = lane * CAP

    wid = lax.axis_index("s") * NC + lax.axis_index("c")

    def scan_bins(k_rem_v, r):
        ncopies = 4 if r == 0 else 1

        def c_body(j, c):
            acc = zero16
            for h in hists[:ncopies]:
                for l in range(16):
                    acc = acc + h[pl.ds(l * 256 + j * 16, 16)]
            tot_v[pl.ds(j * 16, 16)] = acc
            return c
        lax.fori_loop(0, 16, c_body, 0)

        def s_body(jj, carry):
            c_hi_v, bstar_v = carry
            j = 15 - jj
            t = tot_v[pl.ds(j * 16, 16)]
            cs = plsc.cumsum(lax.rev(t, (0,)))
            s_vec = lax.rev(cs, (0,)) + c_hi_v
            s_v[pl.ds(j * 16, 16)] = s_vec
            bins = j * 16 + lane
            cand = jnp.where(s_vec >= k_rem_v, bins, -1)
            return c_hi_v + _bmax(cs), jnp.maximum(bstar_v, _bmax(cand))
        _, bstar_v = lax.fori_loop(0, 16, s_body, (zero16, neg16))
        bstar_v = jnp.maximum(bstar_v, zero16)

        count_eq_v = plsc.load_gather(tot_v, [bstar_v])
        cum_before_v = plsc.load_gather(s_v, [bstar_v]) - count_eq_v
        k_rem2_v = k_rem_v - cum_before_v
        all_eq_v = k_rem2_v == count_eq_v
        return bstar_v, k_rem2_v, all_eq_v

    def route(r, bstar_v, all_eq_v, ns_vec, cnt_vec, nact):
        maxc = jnp.max(cnt_vec)
        if r == 0:
            t_lo_v = (bstar_v << 24) ^ min16
            t_hi_v = ((bstar_v + 1) << 24) ^ min16
            not_top_v = bstar_v < 255

        def r_body(t, carry):
            ns, nk = carry
            pos = bbase + t
            k = plsc.load_gather(au_v, [pos])
            idx = plsc.load_gather(ai_v, [pos])
            valid = cnt_vec > t
            d = _digit_of(k, idx, r)
            if r == 0:
                gt = (k >= t_hi_v) & not_top_v & valid
                eq = (k >= t_lo_v) & jnp.logical_not(gt) & valid
            else:
                gt = (d > bstar_v) & valid
                eq = (d == bstar_v) & valid
                plsc.store_scatter(h0, [lanebase + d], zero16, mask=valid)
            m_sel = gt | (eq & all_eq_v)
            m_keep = eq & jnp.logical_not(all_eq_v)
            pc = plsc.cumsum(m_sel.astype(jnp.int32))
            plsc.store_scatter(su_v, [ns + pc - 1], k, mask=m_sel)
            plsc.store_scatter(si_v, [ns + pc - 1], idx, mask=m_sel)
            plsc.store_scatter(au_v, [bbase + nk], k, mask=m_keep)
            plsc.store_scatter(ai_v, [bbase + nk], idx, mask=m_keep)
            ns = ns + plsc.all_reduce_population_count(m_sel)
            nk = nk + m_keep.astype(jnp.int32)
            return ns, nk
        ns_vec, nk_vec = lax.fori_loop(0, maxc, r_body, (ns_vec, zero16))
        return ns_vec, nk_vec

    def small_round(r, k_rem_v, ns_vec, cnt_vec):
        maxc = jnp.max(cnt_vec)

        def h_body(t, c):
            pos = bbase + t
            k = plsc.load_gather(au_v, [pos])
            idx = plsc.load_gather(ai_v, [pos])
            valid = cnt_vec > t
            d = _digit_of(k, idx, r)
            plsc.addupdate_scatter(h0, [lanebase + d], one16, mask=valid)
            return c
        lax.fori_loop(0, maxc, h_body, 0)

        live = maxc > 0
        bstar_v, k_rem2_v, all_eq_v = scan_bins(k_rem_v, r)
        ns_vec, cnt_vec = route(r, bstar_v, all_eq_v, ns_vec, cnt_vec, None)
        k_rem_v = jnp.where(all_eq_v | jnp.logical_not(live), zero16, k_rem2_v)
        return k_rem_v, ns_vec, cnt_vec

    for rr in range(ROWS_PER_W):
        out_row = wid * ROWS_PER_W + rr
        in_row = (B - 1) - out_row

        pltpu.sync_copy(y_hbm.at[in_row], row_v)

        def z_body(i, c):
            for h in hists:
                h[pl.ds(i * 16, 16)] = zero16
            return c
        lax.fori_loop(0, 256, z_body, 0)

        for q in range(4):
            su_v[pl.ds(q * 16, 16)] = min16
            si_v[pl.ds(q * 16, 16)] = neg16
        ob_v[pl.ds(48, 16)] = neg16

        def h0_body(i, c):
            for u in range(4):
                k = _key_of(row_v[pl.ds(i * 64 + u * 16, 16)])
                d = lax.shift_right_logical(k ^ _MININT, 24)
                plsc.addupdate_scatter(hists[u], [lanebase + d], one16)
            return c
        lax.fori_loop(0, 512, h0_body, 0)

        k_rem_v = jnp.full((16,), K, jnp.int32)
        bstar_v, k_rem2_v, all_eq_v = scan_bins(k_rem_v, 0)
        t_lo_v = (bstar_v << 24) ^ min16

        def b_body(i, carry):
            cnt = carry
            for u in range(4):
                base = i * 64 + u * 16
                k = _key_of(row_v[pl.ds(base, 16)])
                m = k >= t_lo_v
                plsc.store_scatter(au_v, [bbase + cnt], k, mask=m)
                plsc.store_scatter(ai_v, [bbase + cnt], base + lane, mask=m)
                cnt = cnt + m.astype(jnp.int32)
            return cnt
        cnt_vec = lax.fori_loop(0, 512, b_body, zero16)

        ns_vec, cnt_vec = route(0, bstar_v, all_eq_v, zero16, cnt_vec, None)
        k_rem_v = jnp.where(all_eq_v, zero16, k_rem2_v)

        def z0_body(i, c):
            h0[pl.ds(i * 16, 16)] = zero16
            return c
        lax.fori_loop(0, 256, z0_body, 0)

        for r in range(1, 6):
            k_rem_v, ns_vec, cnt_vec = small_round(r, k_rem_v, ns_vec,
                                                   cnt_vec)

        def srt_body(j, carry):
            u0, u1, u2, u3, i0, i1, i2, i3 = carry
            s = _bmax(jnp.maximum(jnp.maximum(u0, u1), jnp.maximum(u2, u3)))
            c0 = jnp.where(u0 == s, i0, -1)
            c1 = jnp.where(u1 == s, i1, -1)
            c2 = jnp.where(u2 == s, i2, -1)
            c3 = jnp.where(u3 == s, i3, -1)
            mi = _bmax(jnp.maximum(jnp.maximum(c0, c1), jnp.maximum(c2, c3)))
            plsc.store_scatter(ob_v, [jnp.full((16,), j, jnp.int32)], mi)
            outs = []
            for (u, iv) in ((u0, i0), (u1, i1), (u2, i2), (u3, i3)):
                hit = (u == s) & (iv == mi)
                outs.append(jnp.where(hit, min16, u))
                outs.append(jnp.where(hit, neg16, iv))
            return (outs[0], outs[2], outs[4], outs[6],
                    outs[1], outs[3], outs[5], outs[7])

        init = (su_v[pl.ds(0, 16)], su_v[pl.ds(16, 16)],
                su_v[pl.ds(32, 16)], su_v[pl.ds(48, 16)],
                si_v[pl.ds(0, 16)], si_v[pl.ds(16, 16)],
                si_v[pl.ds(32, 16)], si_v[pl.ds(48, 16)])
        lax.fori_loop(0, K, srt_body, init)

        pltpu.sync_copy(ob_v, out_hbm.at[out_row])


@functools.cache
def _build_topk_sc():
    return pl.kernel(
        _topk_body,
        out_type=jax.ShapeDtypeStruct((B, OUTW), jnp.int32),
        mesh=plsc.VectorSubcoreMesh(core_axis_name="c", subcore_axis_name="s",
                                    num_cores=NC, num_subcores=NS),
        scratch_types=[
            pltpu.VMEM((N,), jnp.int32),
            pltpu.VMEM((N,), jnp.int32),
            pltpu.VMEM((N,), jnp.int32),
            pltpu.VMEM((16 * 256,), jnp.int32),
            pltpu.VMEM((16 * 256,), jnp.int32),
            pltpu.VMEM((16 * 256,), jnp.int32),
            pltpu.VMEM((16 * 256,), jnp.int32),
            pltpu.VMEM((256,), jnp.int32),
            pltpu.VMEM((256,), jnp.int32),
            pltpu.VMEM((OUTW,), jnp.int32),
            pltpu.VMEM((OUTW,), jnp.int32),
            pltpu.VMEM((OUTW,), jnp.int32),
        ],
        compiler_params=pltpu.CompilerParams(needs_layout_passes=False),
    )


def kernel(y_pred):
    bits = jax.lax.bitcast_convert_type(y_pred[:, -1], jnp.int32)
    return _build_topk_sc()(bits)[:, :K]


# speedup vs baseline: 13.1624x; 1.5095x over previous
import functools

import jax
import jax.numpy as jnp
import numpy as np
from jax import lax
from jax.experimental import pallas as pl
from jax.experimental.pallas import tpu as pltpu
from jax.experimental.pallas import tpu_sc as plsc

B = 64
N = 32768
K = 50
OUTW = 64
NC = 2
NS = 16
NW = NC * NS
ROWS_PER_W = B // NW
CAP = N // 16

_MININT = np.int32(-(2 ** 31))


def _lane():
    return lax.iota(jnp.int32, 16)


def _bmax(x):
    for d in (1, 2, 4, 8):
        x = jnp.maximum(x, jnp.take(x, _lane() ^ d))
    return x


def _key_of(b):
    return b ^ jnp.where(b >= 0, jnp.int32(0), jnp.int32(0x7FFFFFFF))


def _digit_of(k, idx, r):
    ku = k ^ _MININT
    if r == 0:
        return lax.shift_right_logical(ku, 24)
    if r == 1:
        return lax.shift_right_logical(ku, 16) & 0xFF
    if r == 2:
        return lax.shift_right_logical(ku, 8) & 0xFF
    if r == 3:
        return ku & 0xFF
    if r == 4:
        return lax.shift_right_logical(idx, 7)
    return idx & 0x7F


def _topk_body(y_hbm, out_hbm, row_v, au_v, ai_v, h0, h1, h2, h3,
               tot_v, s_v, su_v, si_v, ob_v, dma_sem):
    lane = _lane()
    hists = (h0, h1, h2, h3)
    zero16 = jnp.zeros((16,), jnp.int32)
    one16 = jnp.ones((16,), jnp.int32)
    neg16 = jnp.full((16,), -1, jnp.int32)
    min16 = jnp.full((16,), _MININT, jnp.int32)
    lanebase = lane * 256
    bbase = lane * CAP

    wid = lax.axis_index("s") * NC + lax.axis_index("c")

    def scan_bins(k_rem_v, r):
        ncopies = 4 if r == 0 else 1

        def c_body(j, c):
            acc = zero16
            for h in hists[:ncopies]:
                for l in range(16):
                    acc = acc + h[pl.ds(l * 256 + j * 16, 16)]
            tot_v[pl.ds(j * 16, 16)] = acc
            return c
        lax.fori_loop(0, 16, c_body, 0)

        def s_body(jj, carry):
            c_hi_v, bstar_v = carry
            j = 15 - jj
            t = tot_v[pl.ds(j * 16, 16)]
            cs = plsc.cumsum(lax.rev(t, (0,)))
            s_vec = lax.rev(cs, (0,)) + c_hi_v
            s_v[pl.ds(j * 16, 16)] = s_vec
            bins = j * 16 + lane
            cand = jnp.where(s_vec >= k_rem_v, bins, -1)
            return c_hi_v + _bmax(cs), jnp.maximum(bstar_v, _bmax(cand))
        _, bstar_v = lax.fori_loop(0, 16, s_body, (zero16, neg16))
        bstar_v = jnp.maximum(bstar_v, zero16)

        count_eq_v = plsc.load_gather(tot_v, [bstar_v])
        cum_before_v = plsc.load_gather(s_v, [bstar_v]) - count_eq_v
        k_rem2_v = k_rem_v - cum_before_v
        all_eq_v = k_rem2_v == count_eq_v
        return bstar_v, k_rem2_v, all_eq_v

    def route(r, bstar_v, all_eq_v, ns_vec, cnt_vec, nact):
        maxc = jnp.max(cnt_vec)
        if r == 0:
            t_lo_v = (bstar_v << 24) ^ min16
            t_hi_v = ((bstar_v + 1) << 24) ^ min16
            not_top_v = bstar_v < 255

        def r_body(t, carry):
            ns, nk = carry
            pos = bbase + t
            k = plsc.load_gather(au_v, [pos])
            idx = plsc.load_gather(ai_v, [pos])
            valid = cnt_vec > t
            d = _digit_of(k, idx, r)
            if r == 0:
                gt = (k >= t_hi_v) & not_top_v & valid
                eq = (k >= t_lo_v) & jnp.logical_not(gt) & valid
            else:
                gt = (d > bstar_v) & valid
                eq = (d == bstar_v) & valid
                plsc.store_scatter(h0, [lanebase + d], zero16, mask=valid)
            m_sel = gt | (eq & all_eq_v)
            m_keep = eq & jnp.logical_not(all_eq_v)
            pc = plsc.cumsum(m_sel.astype(jnp.int32))
            plsc.store_scatter(su_v, [ns + pc - 1], k, mask=m_sel)
            plsc.store_scatter(si_v, [ns + pc - 1], idx, mask=m_sel)
            plsc.store_scatter(au_v, [bbase + nk], k, mask=m_keep)
            plsc.store_scatter(ai_v, [bbase + nk], idx, mask=m_keep)
            ns = ns + plsc.all_reduce_population_count(m_sel)
            nk = nk + m_keep.astype(jnp.int32)
            return ns, nk
        ns_vec, nk_vec = lax.fori_loop(0, maxc, r_body, (ns_vec, zero16))
        return ns_vec, nk_vec

    def small_round(r, k_rem_v, ns_vec, cnt_vec):
        maxc = jnp.max(cnt_vec)

        def h_body(t, c):
            pos = bbase + t
            k = plsc.load_gather(au_v, [pos])
            idx = plsc.load_gather(ai_v, [pos])
            valid = cnt_vec > t
            d = _digit_of(k, idx, r)
            plsc.addupdate_scatter(h0, [lanebase + d], one16, mask=valid)
            return c
        lax.fori_loop(0, maxc, h_body, 0)

        live = maxc > 0
        bstar_v, k_rem2_v, all_eq_v = scan_bins(k_rem_v, r)
        ns_vec, cnt_vec = route(r, bstar_v, all_eq_v, ns_vec, cnt_vec, None)
        k_rem_v = jnp.where(all_eq_v | jnp.logical_not(live), zero16, k_rem2_v)
        return k_rem_v, ns_vec, cnt_vec

    for rr in range(ROWS_PER_W):
        out_row = wid * ROWS_PER_W + rr
        in_row = (B - 1) - out_row

        if rr == 0:
            pltpu.sync_copy(y_hbm.at[in_row], row_v)
        else:
            row_dma.wait()

        def z_body(i, c):
            for h in (hists if rr == 0 else hists[1:]):
                h[pl.ds(i * 16, 16)] = zero16
            return c
        lax.fori_loop(0, 256, z_body, 0)

        for q in range(4):
            su_v[pl.ds(q * 16, 16)] = min16
            si_v[pl.ds(q * 16, 16)] = neg16
        ob_v[pl.ds(48, 16)] = neg16

        def h0_body(i, c):
            ks = [_key_of(row_v[pl.ds(i * 128 + u * 16, 16)])
                  for u in range(8)]
            dg = [lax.shift_right_logical(k ^ _MININT, 24) + lanebase
                  for k in ks]
            for u in range(8):
                plsc.addupdate_scatter(hists[u % 4], [dg[u]], one16)
            return c
        lax.fori_loop(0, 256, h0_body, 0)

        k_rem_v = jnp.full((16,), K, jnp.int32)
        bstar_v, k_rem2_v, all_eq_v = scan_bins(k_rem_v, 0)
        t_lo_v = (bstar_v << 24) ^ min16

        def b_body(i, carry):
            cnt = carry
            ks = [_key_of(row_v[pl.ds(i * 128 + u * 16, 16)])
                  for u in range(8)]
            ms = [k >= t_lo_v for k in ks]
            for u in range(8):
                pos = bbase + cnt
                plsc.store_scatter(au_v, [pos], ks[u], mask=ms[u])
                plsc.store_scatter(ai_v, [pos], (i * 128 + u * 16) + lane,
                                   mask=ms[u])
                cnt = cnt + ms[u].astype(jnp.int32)
            return cnt
        cnt_vec = lax.fori_loop(0, 256, b_body, zero16)

        if rr + 1 < ROWS_PER_W:
            row_dma = pltpu.async_copy(
                y_hbm.at[(B - 1) - (out_row + 1)], row_v, dma_sem)

        ns_vec, cnt_vec = route(0, bstar_v, all_eq_v, zero16, cnt_vec, None)
        k_rem_v = jnp.where(all_eq_v, zero16, k_rem2_v)

        def z0_body(i, c):
            h0[pl.ds(i * 16, 16)] = zero16
            return c
        lax.fori_loop(0, 256, z0_body, 0)

        for r in range(1, 6):
            k_rem_v, ns_vec, cnt_vec = small_round(r, k_rem_v, ns_vec,
                                                   cnt_vec)

        def srt_body(j, carry):
            u0, u1, u2, u3, i0, i1, i2, i3 = carry
            s = _bmax(jnp.maximum(jnp.maximum(u0, u1), jnp.maximum(u2, u3)))
            c0 = jnp.where(u0 == s, i0, -1)
            c1 = jnp.where(u1 == s, i1, -1)
            c2 = jnp.where(u2 == s, i2, -1)
            c3 = jnp.where(u3 == s, i3, -1)
            mi = _bmax(jnp.maximum(jnp.maximum(c0, c1), jnp.maximum(c2, c3)))
            plsc.store_scatter(ob_v, [jnp.full((16,), j, jnp.int32)], mi)
            outs = []
            for (u, iv) in ((u0, i0), (u1, i1), (u2, i2), (u3, i3)):
                hit = (u == s) & (iv == mi)
                outs.append(jnp.where(hit, min16, u))
                outs.append(jnp.where(hit, neg16, iv))
            return (outs[0], outs[2], outs[4], outs[6],
                    outs[1], outs[3], outs[5], outs[7])

        init = (su_v[pl.ds(0, 16)], su_v[pl.ds(16, 16)],
                su_v[pl.ds(32, 16)], su_v[pl.ds(48, 16)],
                si_v[pl.ds(0, 16)], si_v[pl.ds(16, 16)],
                si_v[pl.ds(32, 16)], si_v[pl.ds(48, 16)])
        lax.fori_loop(0, K, srt_body, init)

        pltpu.sync_copy(ob_v, out_hbm.at[out_row])


@functools.cache
def _build_topk_sc():
    return pl.kernel(
        _topk_body,
        out_type=jax.ShapeDtypeStruct((B, OUTW), jnp.int32),
        mesh=plsc.VectorSubcoreMesh(core_axis_name="c", subcore_axis_name="s",
                                    num_cores=NC, num_subcores=NS),
        scratch_types=[
            pltpu.VMEM((N,), jnp.int32),
            pltpu.VMEM((N,), jnp.int32),
            pltpu.VMEM((N,), jnp.int32),
            pltpu.VMEM((16 * 256,), jnp.int32),
            pltpu.VMEM((16 * 256,), jnp.int32),
            pltpu.VMEM((16 * 256,), jnp.int32),
            pltpu.VMEM((16 * 256,), jnp.int32),
            pltpu.VMEM((256,), jnp.int32),
            pltpu.VMEM((256,), jnp.int32),
            pltpu.VMEM((OUTW,), jnp.int32),
            pltpu.VMEM((OUTW,), jnp.int32),
            pltpu.VMEM((OUTW,), jnp.int32),
            pltpu.SemaphoreType.DMA,
        ],
        compiler_params=pltpu.CompilerParams(needs_layout_passes=False),
    )


def kernel(y_pred):
    bits = jax.lax.bitcast_convert_type(y_pred[:, -1], jnp.int32)
    return _build_topk_sc()(bits)[:, :K]


# speedup vs baseline: 17.6499x; 1.3409x over previous
import functools

import jax
import jax.numpy as jnp
import numpy as np
from jax import lax
from jax.experimental import pallas as pl
from jax.experimental.pallas import tpu as pltpu
from jax.experimental.pallas import tpu_sc as plsc

B = 64
SEQ = 8
N = 32768
K = 50
OUTW = 64
NC = 2
NS = 16
NW = NC * NS
ROWS_PER_W = B // NW
CAP = N // 16

_MININT = np.int32(-(2 ** 31))


def _lane():
    return lax.iota(jnp.int32, 16)


def _bmax(x):
    for d in (1, 2, 4, 8):
        x = jnp.maximum(x, jnp.take(x, _lane() ^ d))
    return x


def _key_of(b):
    return b ^ jnp.where(b >= 0, jnp.int32(0), jnp.int32(0x7FFFFFFF))


def _digit_of(k, idx, r):
    ku = k ^ _MININT
    if r == 0:
        return lax.shift_right_logical(ku, 24)
    if r == 1:
        return lax.shift_right_logical(ku, 16) & 0xFF
    if r == 2:
        return lax.shift_right_logical(ku, 8) & 0xFF
    if r == 3:
        return ku & 0xFF
    if r == 4:
        return lax.shift_right_logical(idx, 7)
    return idx & 0x7F


def _topk_body(y_hbm, out_hbm, row_v, au_v, ai_v, h0, h1, h2, h3,
               tot_v, s_v, su_v, si_v, ob_v, dma_sem):
    lane = _lane()
    hists = (h0, h1, h2, h3)
    zero16 = jnp.zeros((16,), jnp.int32)
    one16 = jnp.ones((16,), jnp.int32)
    neg16 = jnp.full((16,), -1, jnp.int32)
    min16 = jnp.full((16,), _MININT, jnp.int32)
    lanebase = lane * 256
    bbase = lane * CAP

    wid = lax.axis_index("s") * NC + lax.axis_index("c")

    def scan_bins(k_rem_v, r):
        ncopies = 4 if r == 0 else 1

        def c_body(j, c):
            acc = zero16
            for h in hists[:ncopies]:
                for l in range(16):
                    acc = acc + h[pl.ds(l * 256 + j * 16, 16)]
            tot_v[pl.ds(j * 16, 16)] = acc
            return c
        lax.fori_loop(0, 16, c_body, 0)

        def s_body(jj, carry):
            c_hi_v, bstar_v = carry
            j = 15 - jj
            t = tot_v[pl.ds(j * 16, 16)]
            cs = plsc.cumsum(lax.rev(t, (0,)))
            s_vec = lax.rev(cs, (0,)) + c_hi_v
            s_v[pl.ds(j * 16, 16)] = s_vec
            bins = j * 16 + lane
            cand = jnp.where(s_vec >= k_rem_v, bins, -1)
            return c_hi_v + _bmax(cs), jnp.maximum(bstar_v, _bmax(cand))
        _, bstar_v = lax.fori_loop(0, 16, s_body, (zero16, neg16))
        bstar_v = jnp.maximum(bstar_v, zero16)

        count_eq_v = plsc.load_gather(tot_v, [bstar_v])
        cum_before_v = plsc.load_gather(s_v, [bstar_v]) - count_eq_v
        k_rem2_v = k_rem_v - cum_before_v
        all_eq_v = k_rem2_v == count_eq_v
        return bstar_v, k_rem2_v, all_eq_v

    def route(r, bstar_v, all_eq_v, ns_vec, cnt_vec, nact):
        maxc = jnp.max(cnt_vec)
        if r == 0:
            t_lo_v = (bstar_v << 24) ^ min16
            t_hi_v = ((bstar_v + 1) << 24) ^ min16
            not_top_v = bstar_v < 255

        def r_body(t, carry):
            ns, nk = carry
            pos = bbase + t
            k = plsc.load_gather(au_v, [pos])
            idx = plsc.load_gather(ai_v, [pos])
            valid = cnt_vec > t
            d = _digit_of(k, idx, r)
            if r == 0:
                gt = (k >= t_hi_v) & not_top_v & valid
                eq = (k >= t_lo_v) & jnp.logical_not(gt) & valid
            else:
                gt = (d > bstar_v) & valid
                eq = (d == bstar_v) & valid
                plsc.store_scatter(h0, [lanebase + d], zero16, mask=valid)
            m_sel = gt | (eq & all_eq_v)
            m_keep = eq & jnp.logical_not(all_eq_v)
            pc = plsc.cumsum(m_sel.astype(jnp.int32))
            plsc.store_scatter(su_v, [ns + pc - 1], k, mask=m_sel)
            plsc.store_scatter(si_v, [ns + pc - 1], idx, mask=m_sel)
            plsc.store_scatter(au_v, [bbase + nk], k, mask=m_keep)
            plsc.store_scatter(ai_v, [bbase + nk], idx, mask=m_keep)
            ns = ns + plsc.all_reduce_population_count(m_sel)
            nk = nk + m_keep.astype(jnp.int32)
            return ns, nk
        ns_vec, nk_vec = lax.fori_loop(0, maxc, r_body, (ns_vec, zero16))
        return ns_vec, nk_vec

    def small_round(r, k_rem_v, ns_vec, cnt_vec):
        maxc = jnp.max(cnt_vec)

        def h_body(t, c):
            pos = bbase + t
            k = plsc.load_gather(au_v, [pos])
            idx = plsc.load_gather(ai_v, [pos])
            valid = cnt_vec > t
            d = _digit_of(k, idx, r)
            plsc.addupdate_scatter(h0, [lanebase + d], one16, mask=valid)
            return c
        lax.fori_loop(0, maxc, h_body, 0)

        live = maxc > 0
        bstar_v, k_rem2_v, all_eq_v = scan_bins(k_rem_v, r)
        ns_vec, cnt_vec = route(r, bstar_v, all_eq_v, ns_vec, cnt_vec, None)
        k_rem_v = jnp.where(all_eq_v | jnp.logical_not(live), zero16, k_rem2_v)
        return k_rem_v, ns_vec, cnt_vec

    for rr in range(ROWS_PER_W):
        out_row = wid * ROWS_PER_W + rr
        in_row = (B - 1) - out_row

        if rr == 0:
            pltpu.sync_copy(y_hbm.at[in_row, SEQ - 1], row_v)
        else:
            row_dma.wait()

        def z_body(i, c):
            for h in (hists if rr == 0 else hists[1:]):
                h[pl.ds(i * 16, 16)] = zero16
            return c
        lax.fori_loop(0, 256, z_body, 0)

        for q in range(4):
            su_v[pl.ds(q * 16, 16)] = min16
            si_v[pl.ds(q * 16, 16)] = neg16
        ob_v[pl.ds(48, 16)] = neg16

        def h0_body(i, c):
            ks = [_key_of(plsc.bitcast(row_v[pl.ds(i * 128 + u * 16, 16)],
                                       jnp.int32))
                  for u in range(8)]
            dg = [lax.shift_right_logical(k ^ _MININT, 24) + lanebase
                  for k in ks]
            for u in range(8):
                plsc.addupdate_scatter(hists[u % 4], [dg[u]], one16)
            return c
        lax.fori_loop(0, 256, h0_body, 0)

        k_rem_v = jnp.full((16,), K, jnp.int32)
        bstar_v, k_rem2_v, all_eq_v = scan_bins(k_rem_v, 0)
        t_lo_v = (bstar_v << 24) ^ min16

        def b_body(i, carry):
            cnt = carry
            ks = [_key_of(plsc.bitcast(row_v[pl.ds(i * 128 + u * 16, 16)],
                                       jnp.int32))
                  for u in range(8)]
            ms = [k >= t_lo_v for k in ks]
            for u in range(8):
                pos = bbase + cnt
                plsc.store_scatter(au_v, [pos], ks[u], mask=ms[u])
                plsc.store_scatter(ai_v, [pos], (i * 128 + u * 16) + lane,
                                   mask=ms[u])
                cnt = cnt + ms[u].astype(jnp.int32)
            return cnt
        cnt_vec = lax.fori_loop(0, 256, b_body, zero16)

        if rr + 1 < ROWS_PER_W:
            row_dma = pltpu.async_copy(
                y_hbm.at[(B - 1) - (out_row + 1), SEQ - 1], row_v, dma_sem)

        ns_vec, cnt_vec = route(0, bstar_v, all_eq_v, zero16, cnt_vec, None)
        k_rem_v = jnp.where(all_eq_v, zero16, k_rem2_v)

        def z0_body(i, c):
            h0[pl.ds(i * 16, 16)] = zero16
            return c
        lax.fori_loop(0, 256, z0_body, 0)

        for r in range(1, 6):
            k_rem_v, ns_vec, cnt_vec = small_round(r, k_rem_v, ns_vec,
                                                   cnt_vec)

        def srt_body(j, carry):
            u0, u1, u2, u3, i0, i1, i2, i3 = carry
            s = _bmax(jnp.maximum(jnp.maximum(u0, u1), jnp.maximum(u2, u3)))
            c0 = jnp.where(u0 == s, i0, -1)
            c1 = jnp.where(u1 == s, i1, -1)
            c2 = jnp.where(u2 == s, i2, -1)
            c3 = jnp.where(u3 == s, i3, -1)
            mi = _bmax(jnp.maximum(jnp.maximum(c0, c1), jnp.maximum(c2, c3)))
            plsc.store_scatter(ob_v, [jnp.full((16,), j, jnp.int32)], mi)
            outs = []
            for (u, iv) in ((u0, i0), (u1, i1), (u2, i2), (u3, i3)):
                hit = (u == s) & (iv == mi)
                outs.append(jnp.where(hit, min16, u))
                outs.append(jnp.where(hit, neg16, iv))
            return (outs[0], outs[2], outs[4], outs[6],
                    outs[1], outs[3], outs[5], outs[7])

        init = (su_v[pl.ds(0, 16)], su_v[pl.ds(16, 16)],
                su_v[pl.ds(32, 16)], su_v[pl.ds(48, 16)],
                si_v[pl.ds(0, 16)], si_v[pl.ds(16, 16)],
                si_v[pl.ds(32, 16)], si_v[pl.ds(48, 16)])
        lax.fori_loop(0, K, srt_body, init)

        pltpu.sync_copy(ob_v, out_hbm.at[out_row])


@functools.cache
def _build_topk_sc():
    return pl.kernel(
        _topk_body,
        name="topk_radix_select",
        out_type=jax.ShapeDtypeStruct((B, OUTW), jnp.int32),
        mesh=plsc.VectorSubcoreMesh(core_axis_name="c", subcore_axis_name="s",
                                    num_cores=NC, num_subcores=NS),
        scratch_types=[
            pltpu.VMEM((N,), jnp.float32),
            pltpu.VMEM((N,), jnp.int32),
            pltpu.VMEM((N,), jnp.int32),
            pltpu.VMEM((16 * 256,), jnp.int32),
            pltpu.VMEM((16 * 256,), jnp.int32),
            pltpu.VMEM((16 * 256,), jnp.int32),
            pltpu.VMEM((16 * 256,), jnp.int32),
            pltpu.VMEM((256,), jnp.int32),
            pltpu.VMEM((256,), jnp.int32),
            pltpu.VMEM((OUTW,), jnp.int32),
            pltpu.VMEM((OUTW,), jnp.int32),
            pltpu.VMEM((OUTW,), jnp.int32),
            pltpu.SemaphoreType.DMA,
        ],
        compiler_params=pltpu.CompilerParams(needs_layout_passes=False),
    )


def kernel(y_pred):
    return _build_topk_sc()(y_pred)[:, :K]


# speedup vs baseline: 18.2414x; 1.0335x over previous
import functools

import jax
import jax.numpy as jnp
import numpy as np
from jax import lax
from jax.experimental import pallas as pl
from jax.experimental.pallas import tpu as pltpu
from jax.experimental.pallas import tpu_sc as plsc

B = 64
SEQ = 8
N = 32768
K = 50
OUTW = 64
NC = 2
NS = 16
NW = NC * NS
ROWS_PER_W = B // NW
CAP = N // 16

_MININT = np.int32(-(2 ** 31))


def _lane():
    return lax.iota(jnp.int32, 16)


def _bmax(x):
    for d in (1, 2, 4, 8):
        x = jnp.maximum(x, jnp.take(x, _lane() ^ d))
    return x


def _bsum(x):
    for d in (1, 2, 4, 8):
        x = x + jnp.take(x, _lane() ^ d)
    return x


def _key_of(b):
    return b ^ jnp.where(b >= 0, jnp.int32(0), jnp.int32(0x7FFFFFFF))


def _digit_of(k, idx, r):
    ku = k ^ _MININT
    if r == 0:
        return lax.shift_right_logical(ku, 24)
    if r == 1:
        return lax.shift_right_logical(ku, 16) & 0xFF
    if r == 2:
        return lax.shift_right_logical(ku, 8) & 0xFF
    if r == 3:
        return ku & 0xFF
    if r == 4:
        return lax.shift_right_logical(idx, 7)
    return idx & 0x7F


def _topk_body(y_hbm, out_hbm, row_v, au_v, ai_v, h0, h1, h2, h3,
               tot_v, s_v, su_v, si_v, ob_v, dma_sem):
    lane = _lane()
    hists = (h0, h1, h2, h3)
    zero16 = jnp.zeros((16,), jnp.int32)
    one16 = jnp.ones((16,), jnp.int32)
    neg16 = jnp.full((16,), -1, jnp.int32)
    min16 = jnp.full((16,), _MININT, jnp.int32)
    lanebase = lane * 256
    bbase = lane * CAP

    wid = lax.axis_index("s") * NC + lax.axis_index("c")

    def scan_bins(k_rem_v, r, n16):
        ncopies = 4 if r == 0 else 1

        def c_body(j, c):
            acc = zero16
            for h in hists[:ncopies]:
                for l in range(16):
                    acc = acc + h[pl.ds(l * 256 + j * 16, 16)]
            tot_v[pl.ds(j * 16, 16)] = acc
            return c
        lax.fori_loop(0, n16, c_body, 0)

        def s_body(jj, carry):
            c_hi_v, bstar_v = carry
            j = 15 - jj
            t = tot_v[pl.ds(j * 16, 16)]
            cs = plsc.cumsum(lax.rev(t, (0,)))
            s_vec = lax.rev(cs, (0,)) + c_hi_v
            s_v[pl.ds(j * 16, 16)] = s_vec
            bins = j * 16 + lane
            cand = jnp.where(s_vec >= k_rem_v, bins, -1)
            return c_hi_v + _bmax(cs), jnp.maximum(bstar_v, _bmax(cand))
        _, bstar_v = lax.fori_loop(0, n16, s_body, (zero16, neg16))
        bstar_v = jnp.maximum(bstar_v, zero16)

        count_eq_v = plsc.load_gather(tot_v, [bstar_v])
        cum_before_v = plsc.load_gather(s_v, [bstar_v]) - count_eq_v
        k_rem2_v = k_rem_v - cum_before_v
        all_eq_v = k_rem2_v == count_eq_v
        return bstar_v, k_rem2_v, all_eq_v

    def route(r, bstar_v, all_eq_v, ns_vec, cnt_vec, nact):
        maxc = jnp.max(cnt_vec)
        if r == 0:
            t_lo_v = (bstar_v << 24) ^ min16
            t_hi_v = ((bstar_v + 1) << 24) ^ min16
            not_top_v = bstar_v < 255

        def r_body(t, carry):
            ns, nk = carry
            pos = bbase + t
            k = plsc.load_gather(au_v, [pos])
            if r == 0:
                k = _key_of(k)
            idx = plsc.load_gather(ai_v, [pos])
            valid = cnt_vec > t
            d = _digit_of(k, idx, r)
            if r == 0:
                gt = (k >= t_hi_v) & not_top_v & valid
                eq = (k >= t_lo_v) & jnp.logical_not(gt) & valid
            else:
                gt = (d > bstar_v) & valid
                eq = (d == bstar_v) & valid
                plsc.store_scatter(h0, [lanebase + d], zero16, mask=valid)
            m_sel = gt | (eq & all_eq_v)
            m_keep = eq & jnp.logical_not(all_eq_v)
            pc = plsc.cumsum(m_sel.astype(jnp.int32))
            plsc.store_scatter(su_v, [ns + pc - 1], k, mask=m_sel)
            plsc.store_scatter(si_v, [ns + pc - 1], idx, mask=m_sel)
            plsc.store_scatter(au_v, [bbase + nk], k, mask=m_keep)
            plsc.store_scatter(ai_v, [bbase + nk], idx, mask=m_keep)
            ns = ns + plsc.all_reduce_population_count(m_sel)
            nk = nk + m_keep.astype(jnp.int32)
            return ns, nk
        ns_vec, nk_vec = lax.fori_loop(0, maxc, r_body, (ns_vec, zero16))
        return ns_vec, nk_vec

    def small_round(r, k_rem_v, ns_vec, cnt_vec):
        can_dump_v = (ns_vec + _bsum(cnt_vec)) <= OUTW
        dump_s = jnp.max(jnp.where(can_dump_v, 1, 0)) == 1
        maxc = jnp.max(cnt_vec)

        def d_body(t, ns):
            pos = bbase + t
            k = plsc.load_gather(au_v, [pos])
            idx = plsc.load_gather(ai_v, [pos])
            valid = cnt_vec > t
            pc = plsc.cumsum(valid.astype(jnp.int32))
            plsc.store_scatter(su_v, [ns + pc - 1], k, mask=valid)
            plsc.store_scatter(si_v, [ns + pc - 1], idx, mask=valid)
            return ns + plsc.all_reduce_population_count(valid)
        ns_vec = lax.fori_loop(0, jnp.where(dump_s, maxc, 0), d_body, ns_vec)
        cnt_vec = jnp.where(can_dump_v, zero16, cnt_vec)
        maxc = jnp.where(dump_s, 0, maxc)
        live = maxc > 0
        n16 = jnp.where(live, jnp.int32(16), jnp.int32(0))

        def h_body(t, c):
            pos = bbase + t
            k = plsc.load_gather(au_v, [pos])
            idx = plsc.load_gather(ai_v, [pos])
            valid = cnt_vec > t
            d = _digit_of(k, idx, r)
            plsc.addupdate_scatter(h0, [lanebase + d], one16, mask=valid)
            return c
        lax.fori_loop(0, maxc, h_body, 0)

        bstar_v, k_rem2_v, all_eq_v = scan_bins(k_rem_v, r, n16)
        ns_vec, cnt_vec = route(r, bstar_v, all_eq_v, ns_vec, cnt_vec, None)
        k_rem_v = jnp.where(all_eq_v | jnp.logical_not(live), zero16, k_rem2_v)
        return k_rem_v, ns_vec, cnt_vec

    for rr in range(ROWS_PER_W):
        out_row = wid * ROWS_PER_W + rr
        in_row = (B - 1) - out_row

        if rr == 0:
            pltpu.sync_copy(y_hbm.at[in_row, SEQ - 1], row_v)
        else:
            row_dma.wait()

        def z_body(i, c):
            for h in (hists if rr == 0 else hists[1:]):
                h[pl.ds(i * 16, 16)] = zero16
            return c
        lax.fori_loop(0, 256, z_body, 0)

        for q in range(4):
            su_v[pl.ds(q * 16, 16)] = min16
            si_v[pl.ds(q * 16, 16)] = neg16
        ob_v[pl.ds(48, 16)] = neg16

        def h0_body(i, c):
            bs = [plsc.bitcast(row_v[pl.ds(i * 128 + u * 16, 16)], jnp.int32)
                  for u in range(8)]
            dg = [lax.shift_right_logical(
                      b ^ (lax.shift_right_arithmetic(b, 31) | _MININT), 24)
                  + lanebase
                  for b in bs]
            for u in range(8):
                plsc.addupdate_scatter(hists[u % 4], [dg[u]], one16)
            return c
        lax.fori_loop(0, 256, h0_body, 0)

        k_rem_v = jnp.full((16,), K, jnp.int32)
        bstar_v, k_rem2_v, all_eq_v = scan_bins(k_rem_v, 0, jnp.int32(16))
        t_lo_v = (bstar_v << 24) ^ min16
        t_bits_v = jnp.where(t_lo_v >= 0, t_lo_v,
                             t_lo_v ^ jnp.int32(0x7FFFFFFF))
        t_f_v = plsc.bitcast(t_bits_v, jnp.float32)

        def b_body(i, carry):
            cnt = carry
            vs = [row_v[pl.ds(i * 128 + u * 16, 16)] for u in range(8)]
            ms = [v >= t_f_v for v in vs]
            for u in range(8):
                pos = bbase + cnt
                plsc.store_scatter(au_v, [pos],
                                   plsc.bitcast(vs[u], jnp.int32),
                                   mask=ms[u])
                plsc.store_scatter(ai_v, [pos], (i * 128 + u * 16) + lane,
                                   mask=ms[u])
                cnt = cnt + ms[u].astype(jnp.int32)
            return cnt
        cnt_vec = lax.fori_loop(0, 256, b_body, zero16)

        if rr + 1 < ROWS_PER_W:
            row_dma = pltpu.async_copy(
                y_hbm.at[(B - 1) - (out_row + 1), SEQ - 1], row_v, dma_sem)

        ns_vec, cnt_vec = route(0, bstar_v, all_eq_v, zero16, cnt_vec, None)
        k_rem_v = jnp.where(all_eq_v, zero16, k_rem2_v)

        def z0_body(i, c):
            h0[pl.ds(i * 16, 16)] = zero16
            return c
        lax.fori_loop(0, 256, z0_body, 0)

        for r in range(1, 6):
            k_rem_v, ns_vec, cnt_vec = small_round(r, k_rem_v, ns_vec,
                                                   cnt_vec)

        def srt_body(j, carry):
            u0, u1, u2, u3, i0, i1, i2, i3 = carry
            s = _bmax(jnp.maximum(jnp.maximum(u0, u1), jnp.maximum(u2, u3)))
            c0 = jnp.where(u0 == s, i0, -1)
            c1 = jnp.where(u1 == s, i1, -1)
            c2 = jnp.where(u2 == s, i2, -1)
            c3 = jnp.where(u3 == s, i3, -1)
            mi = _bmax(jnp.maximum(jnp.maximum(c0, c1), jnp.maximum(c2, c3)))
            plsc.store_scatter(ob_v, [jnp.full((16,), j, jnp.int32)], mi)
            outs = []
            for (u, iv) in ((u0, i0), (u1, i1), (u2, i2), (u3, i3)):
                hit = (u == s) & (iv == mi)
                outs.append(jnp.where(hit, min16, u))
                outs.append(jnp.where(hit, neg16, iv))
            return (outs[0], outs[2], outs[4], outs[6],
                    outs[1], outs[3], outs[5], outs[7])

        init = (su_v[pl.ds(0, 16)], su_v[pl.ds(16, 16)],
                su_v[pl.ds(32, 16)], su_v[pl.ds(48, 16)],
                si_v[pl.ds(0, 16)], si_v[pl.ds(16, 16)],
                si_v[pl.ds(32, 16)], si_v[pl.ds(48, 16)])
        lax.fori_loop(0, K, srt_body, init)

        pltpu.sync_copy(ob_v, out_hbm.at[out_row])


@functools.cache
def _build_topk_sc():
    return pl.kernel(
        _topk_body,
        name="topk_radix_select",
        out_type=jax.ShapeDtypeStruct((B, OUTW), jnp.int32),
        mesh=plsc.VectorSubcoreMesh(core_axis_name="c", subcore_axis_name="s",
                                    num_cores=NC, num_subcores=NS),
        scratch_types=[
            pltpu.VMEM((N,), jnp.float32),
            pltpu.VMEM((N,), jnp.int32),
            pltpu.VMEM((N,), jnp.int32),
            pltpu.VMEM((16 * 256,), jnp.int32),
            pltpu.VMEM((16 * 256,), jnp.int32),
            pltpu.VMEM((16 * 256,), jnp.int32),
            pltpu.VMEM((16 * 256,), jnp.int32),
            pltpu.VMEM((256,), jnp.int32),
            pltpu.VMEM((256,), jnp.int32),
            pltpu.VMEM((OUTW,), jnp.int32),
            pltpu.VMEM((OUTW,), jnp.int32),
            pltpu.VMEM((OUTW,), jnp.int32),
            pltpu.SemaphoreType.DMA,
        ],
        compiler_params=pltpu.CompilerParams(needs_layout_passes=False),
    )


def kernel(y_pred):
    return _build_topk_sc()(y_pred)[:, :K]


# speedup vs baseline: 19.1366x; 1.0491x over previous
import functools

import jax
import jax.numpy as jnp
import numpy as np
from jax import lax
from jax.experimental import pallas as pl
from jax.experimental.pallas import tpu as pltpu
from jax.experimental.pallas import tpu_sc as plsc

B = 64
SEQ = 8
N = 32768
K = 50
OUTW = 64
NC = 2
NS = 16
NW = NC * NS
ROWS_PER_W = B // NW
CAP = N // 16

_MININT = np.int32(-(2 ** 31))


def _lane():
    return lax.iota(jnp.int32, 16)


def _bmax(x):
    for d in (1, 2, 4, 8):
        x = jnp.maximum(x, jnp.take(x, _lane() ^ d))
    return x


def _bsum(x):
    for d in (1, 2, 4, 8):
        x = x + jnp.take(x, _lane() ^ d)
    return x


def _key_of(b):
    return b ^ jnp.where(b >= 0, jnp.int32(0), jnp.int32(0x7FFFFFFF))


def _digit_of(k, idx, r):
    ku = k ^ _MININT
    if r == 0:
        return lax.shift_right_logical(ku, 24)
    if r == 1:
        return lax.shift_right_logical(ku, 16) & 0xFF
    if r == 2:
        return lax.shift_right_logical(ku, 8) & 0xFF
    if r == 3:
        return ku & 0xFF
    if r == 4:
        return lax.shift_right_logical(idx, 7)
    return idx & 0x7F


def _topk_body(y_hbm, out_hbm, row_v, au_v, ai_v, h0, h1, h2, h3,
               tot_v, s_v, su_v, si_v, ob_v, dma_sem):
    lane = _lane()
    hists = (h0, h1, h2, h3)
    zero16 = jnp.zeros((16,), jnp.int32)
    one16 = jnp.ones((16,), jnp.int32)
    neg16 = jnp.full((16,), -1, jnp.int32)
    min16 = jnp.full((16,), _MININT, jnp.int32)
    lanebase = lane * 256
    bbase = lane * CAP

    wid = lax.axis_index("s") * NC + lax.axis_index("c")

    def scan_bins(k_rem_v, r, n16):
        ncopies = 4 if r == 0 else 1

        def c_body(j, c):
            acc = zero16
            for h in hists[:ncopies]:
                for l in range(16):
                    acc = acc + h[pl.ds(l * 256 + j * 16, 16)]
            tot_v[pl.ds(j * 16, 16)] = acc
            return c
        lax.fori_loop(0, n16, c_body, 0)

        def s_body(jj, carry):
            c_hi_v, bstar_v = carry
            j = 15 - jj
            t = tot_v[pl.ds(j * 16, 16)]
            cs = plsc.cumsum(lax.rev(t, (0,)))
            s_vec = lax.rev(cs, (0,)) + c_hi_v
            s_v[pl.ds(j * 16, 16)] = s_vec
            bins = j * 16 + lane
            cand = jnp.where(s_vec >= k_rem_v, bins, -1)
            return c_hi_v + _bmax(cs), jnp.maximum(bstar_v, _bmax(cand))
        _, bstar_v = lax.fori_loop(0, n16, s_body, (zero16, neg16))
        bstar_v = jnp.maximum(bstar_v, zero16)

        count_eq_v = plsc.load_gather(tot_v, [bstar_v])
        cum_before_v = plsc.load_gather(s_v, [bstar_v]) - count_eq_v
        k_rem2_v = k_rem_v - cum_before_v
        all_eq_v = k_rem2_v == count_eq_v
        return bstar_v, k_rem2_v, all_eq_v

    def route(r, bstar_v, all_eq_v, ns_vec, cnt_vec, nact):
        maxc = jnp.max(cnt_vec)
        if r == 0:
            t_lo_v = (bstar_v << 24) ^ min16
            t_hi_v = ((bstar_v + 1) << 24) ^ min16
            not_top_v = bstar_v < 255

        def r_body(t, carry):
            ns, nk = carry
            pos = bbase + t
            k = plsc.load_gather(au_v, [pos])
            if r == 0:
                k = _key_of(k)
            idx = plsc.load_gather(ai_v, [pos])
            valid = cnt_vec > t
            d = _digit_of(k, idx, r)
            if r == 0:
                gt = (k >= t_hi_v) & not_top_v & valid
                eq = (k >= t_lo_v) & jnp.logical_not(gt) & valid
            else:
                gt = (d > bstar_v) & valid
                eq = (d == bstar_v) & valid
                plsc.store_scatter(h0, [lanebase + d], zero16, mask=valid)
            m_sel = gt | (eq & all_eq_v)
            m_keep = eq & jnp.logical_not(all_eq_v)
            pc = plsc.cumsum(m_sel.astype(jnp.int32))
            plsc.store_scatter(su_v, [ns + pc - 1], k, mask=m_sel)
            plsc.store_scatter(si_v, [ns + pc - 1], idx, mask=m_sel)
            plsc.store_scatter(au_v, [bbase + nk], k, mask=m_keep)
            plsc.store_scatter(ai_v, [bbase + nk], idx, mask=m_keep)
            ns = ns + plsc.all_reduce_population_count(m_sel)
            nk = nk + m_keep.astype(jnp.int32)
            return ns, nk
        ns_vec, nk_vec = lax.fori_loop(0, maxc, r_body, (ns_vec, zero16))
        return ns_vec, nk_vec

    def small_round(r, k_rem_v, ns_vec, cnt_vec):
        can_dump_v = (ns_vec + _bsum(cnt_vec)) <= OUTW
        dump_s = jnp.max(jnp.where(can_dump_v, 1, 0)) == 1
        maxc = jnp.max(cnt_vec)

        def d_body(t, ns):
            pos = bbase + t
            k = plsc.load_gather(au_v, [pos])
            idx = plsc.load_gather(ai_v, [pos])
            valid = cnt_vec > t
            pc = plsc.cumsum(valid.astype(jnp.int32))
            plsc.store_scatter(su_v, [ns + pc - 1], k, mask=valid)
            plsc.store_scatter(si_v, [ns + pc - 1], idx, mask=valid)
            return ns + plsc.all_reduce_population_count(valid)
        ns_vec = lax.fori_loop(0, jnp.where(dump_s, maxc, 0), d_body, ns_vec)
        cnt_vec = jnp.where(can_dump_v, zero16, cnt_vec)
        maxc = jnp.where(dump_s, 0, maxc)
        live = maxc > 0
        n16 = jnp.where(live, jnp.int32(16), jnp.int32(0))

        def h_body(t, c):
            pos = bbase + t
            k = plsc.load_gather(au_v, [pos])
            idx = plsc.load_gather(ai_v, [pos])
            valid = cnt_vec > t
            d = _digit_of(k, idx, r)
            plsc.addupdate_scatter(h0, [lanebase + d], one16, mask=valid)
            return c
        lax.fori_loop(0, maxc, h_body, 0)

        bstar_v, k_rem2_v, all_eq_v = scan_bins(k_rem_v, r, n16)
        ns_vec, cnt_vec = route(r, bstar_v, all_eq_v, ns_vec, cnt_vec, None)
        k_rem_v = jnp.where(all_eq_v | jnp.logical_not(live), zero16, k_rem2_v)
        return k_rem_v, ns_vec, cnt_vec

    for rr in range(ROWS_PER_W):
        out_row = wid * ROWS_PER_W + rr
        in_row = (B - 1) - out_row

        if rr == 0:
            pltpu.sync_copy(y_hbm.at[in_row, SEQ - 1], row_v)
        else:
            row_dma.wait()

        if rr == 0:
            def z_body(i, c):
                for h in hists:
                    h[pl.ds(i * 16, 16)] = zero16
                return c
            lax.fori_loop(0, 256, z_body, 0)

        for q in range(4):
            su_v[pl.ds(q * 16, 16)] = min16
            si_v[pl.ds(q * 16, 16)] = neg16
        ob_v[pl.ds(48, 16)] = neg16

        def h0_body(i, c):
            bs = [plsc.bitcast(row_v[pl.ds(i * 256 + u * 16, 16)], jnp.int32)
                  for u in range(16)]
            dg = [lax.shift_right_logical(
                      b ^ (lax.shift_right_arithmetic(b, 31) | _MININT), 24)
                  + lanebase
                  for b in bs]
            for u in range(16):
                plsc.addupdate_scatter(hists[u % 4], [dg[u]], one16)
            return c
        lax.fori_loop(0, 128, h0_body, 0)

        k_rem_v = jnp.full((16,), K, jnp.int32)
        bstar_v, k_rem2_v, all_eq_v = scan_bins(k_rem_v, 0, jnp.int32(16))
        t_lo_v = (bstar_v << 24) ^ min16
        t_bits_v = jnp.where(t_lo_v >= 0, t_lo_v,
                             t_lo_v ^ jnp.int32(0x7FFFFFFF))
        t_f_v = plsc.bitcast(t_bits_v, jnp.float32)

        def b_body(i, carry):
            cnt = carry
            vs = [row_v[pl.ds(i * 128 + u * 16, 16)] for u in range(8)]
            ms = [v >= t_f_v for v in vs]
            for u in range(8):
                pos = bbase + cnt
                plsc.store_scatter(au_v, [pos],
                                   plsc.bitcast(vs[u], jnp.int32),
                                   mask=ms[u])
                plsc.store_scatter(ai_v, [pos], (i * 128 + u * 16) + lane,
                                   mask=ms[u])
                cnt = cnt + ms[u].astype(jnp.int32)
            for h in hists:
                h[pl.ds(i * 16, 16)] = zero16
            return cnt
        cnt_vec = lax.fori_loop(0, 256, b_body, zero16)

        if rr + 1 < ROWS_PER_W:
            row_dma = pltpu.async_copy(
                y_hbm.at[(B - 1) - (out_row + 1), SEQ - 1], row_v, dma_sem)

        ns_vec, cnt_vec = route(0, bstar_v, all_eq_v, zero16, cnt_vec, None)
        k_rem_v = jnp.where(all_eq_v, zero16, k_rem2_v)

        for r in range(1, 6):
            k_rem_v, ns_vec, cnt_vec = small_round(r, k_rem_v, ns_vec,
                                                   cnt_vec)

        def srt_body(j, carry):
            u0, u1, u2, u3, i0, i1, i2, i3 = carry
            s = _bmax(jnp.maximum(jnp.maximum(u0, u1), jnp.maximum(u2, u3)))
            c0 = jnp.where(u0 == s, i0, -1)
            c1 = jnp.where(u1 == s, i1, -1)
            c2 = jnp.where(u2 == s, i2, -1)
            c3 = jnp.where(u3 == s, i3, -1)
            mi = _bmax(jnp.maximum(jnp.maximum(c0, c1), jnp.maximum(c2, c3)))
            plsc.store_scatter(ob_v, [jnp.full((16,), j, jnp.int32)], mi)
            outs = []
            for (u, iv) in ((u0, i0), (u1, i1), (u2, i2), (u3, i3)):
                hit = (u == s) & (iv == mi)
                outs.append(jnp.where(hit, min16, u))
                outs.append(jnp.where(hit, neg16, iv))
            return (outs[0], outs[2], outs[4], outs[6],
                    outs[1], outs[3], outs[5], outs[7])

        init = (su_v[pl.ds(0, 16)], su_v[pl.ds(16, 16)],
                su_v[pl.ds(32, 16)], su_v[pl.ds(48, 16)],
                si_v[pl.ds(0, 16)], si_v[pl.ds(16, 16)],
                si_v[pl.ds(32, 16)], si_v[pl.ds(48, 16)])
        lax.fori_loop(0, K, srt_body, init)

        pltpu.sync_copy(ob_v, out_hbm.at[out_row])


@functools.cache
def _build_topk_sc():
    return pl.kernel(
        _topk_body,
        name="topk_radix_select",
        out_type=jax.ShapeDtypeStruct((B, OUTW), jnp.int32),
        mesh=plsc.VectorSubcoreMesh(core_axis_name="c", subcore_axis_name="s",
                                    num_cores=NC, num_subcores=NS),
        scratch_types=[
            pltpu.VMEM((N,), jnp.float32),
            pltpu.VMEM((N,), jnp.int32),
            pltpu.VMEM((N,), jnp.int32),
            pltpu.VMEM((16 * 256,), jnp.int32),
            pltpu.VMEM((16 * 256,), jnp.int32),
            pltpu.VMEM((16 * 256,), jnp.int32),
            pltpu.VMEM((16 * 256,), jnp.int32),
            pltpu.VMEM((256,), jnp.int32),
            pltpu.VMEM((256,), jnp.int32),
            pltpu.VMEM((OUTW,), jnp.int32),
            pltpu.VMEM((OUTW,), jnp.int32),
            pltpu.VMEM((OUTW,), jnp.int32),
            pltpu.SemaphoreType.DMA,
        ],
        compiler_params=pltpu.CompilerParams(needs_layout_passes=False),
    )


def kernel(y_pred):
    return _build_topk_sc()(y_pred)[:, :K]


# speedup vs baseline: 19.3930x; 1.0134x over previous
import functools

import jax
import jax.numpy as jnp
import numpy as np
from jax import lax
from jax.experimental import pallas as pl
from jax.experimental.pallas import tpu as pltpu
from jax.experimental.pallas import tpu_sc as plsc

B = 64
SEQ = 8
N = 32768
K = 50
OUTW = 64
NC = 2
NS = 16
NW = NC * NS
ROWS_PER_W = B // NW
CAP = N // 16

_MININT = np.int32(-(2 ** 31))


def _lane():
    return lax.iota(jnp.int32, 16)


def _bmax(x):
    for d in (1, 2, 4, 8):
        x = jnp.maximum(x, jnp.take(x, _lane() ^ d))
    return x


def _bsum(x):
    for d in (1, 2, 4, 8):
        x = x + jnp.take(x, _lane() ^ d)
    return x


def _key_of(b):
    return b ^ jnp.where(b >= 0, jnp.int32(0), jnp.int32(0x7FFFFFFF))


def _digit_of(k, idx, r):
    ku = k ^ _MININT
    if r == 0:
        return lax.shift_right_logical(ku, 24)
    if r == 1:
        return lax.shift_right_logical(ku, 16) & 0xFF
    if r == 2:
        return lax.shift_right_logical(ku, 8) & 0xFF
    if r == 3:
        return ku & 0xFF
    if r == 4:
        return lax.shift_right_logical(idx, 7)
    return idx & 0x7F


def _topk_body(y_hbm, out_hbm, row_v, au_v, ai_v, h0, h1, h2, h3,
               tot_v, s_v, su_v, si_v, ob_v, dma_sem):
    lane = _lane()
    hists = (h0, h1, h2, h3)
    zero16 = jnp.zeros((16,), jnp.int32)
    one16 = jnp.ones((16,), jnp.int32)
    neg16 = jnp.full((16,), -1, jnp.int32)
    min16 = jnp.full((16,), _MININT, jnp.int32)
    lanebase = lane * 256
    bbase = lane * CAP

    wid = lax.axis_index("s") * NC + lax.axis_index("c")

    def scan_bins(k_rem_v, r, n16):
        ncopies = 4 if r == 0 else 1

        def c_body(j, c):
            acc = zero16
            for h in hists[:ncopies]:
                for l in range(16):
                    acc = acc + h[pl.ds(l * 256 + j * 16, 16)]
            tot_v[pl.ds(j * 16, 16)] = acc
            return c
        lax.fori_loop(0, n16, c_body, 0)

        def s_body(jj, carry):
            c_hi_v, bstar_v = carry
            j = 15 - jj
            t = tot_v[pl.ds(j * 16, 16)]
            cs = plsc.cumsum(lax.rev(t, (0,)))
            s_vec = lax.rev(cs, (0,)) + c_hi_v
            s_v[pl.ds(j * 16, 16)] = s_vec
            bins = j * 16 + lane
            cand = jnp.where(s_vec >= k_rem_v, bins, -1)
            return c_hi_v + _bmax(cs), jnp.maximum(bstar_v, _bmax(cand))
        _, bstar_v = lax.fori_loop(0, n16, s_body, (zero16, neg16))
        bstar_v = jnp.maximum(bstar_v, zero16)

        count_eq_v = plsc.load_gather(tot_v, [bstar_v])
        cum_before_v = plsc.load_gather(s_v, [bstar_v]) - count_eq_v
        k_rem2_v = k_rem_v - cum_before_v
        all_eq_v = k_rem2_v == count_eq_v
        return bstar_v, k_rem2_v, all_eq_v

    def route(r, bstar_v, all_eq_v, ns_vec, cnt_vec, nact):
        maxc = jnp.max(cnt_vec)
        if r == 0:
            t_lo_v = (bstar_v << 24) ^ min16
            t_hi_v = ((bstar_v + 1) << 24) ^ min16
            not_top_v = bstar_v < 255

        def r_body(t, carry):
            ns, nk = carry
            pos = bbase + t
            k = plsc.load_gather(au_v, [pos])
            if r == 0:
                k = _key_of(k)
            idx = plsc.load_gather(ai_v, [pos])
            valid = cnt_vec > t
            d = _digit_of(k, idx, r)
            if r == 0:
                gt = (k >= t_hi_v) & not_top_v & valid
                eq = (k >= t_lo_v) & jnp.logical_not(gt) & valid
            else:
                gt = (d > bstar_v) & valid
                eq = (d == bstar_v) & valid
                plsc.store_scatter(h0, [lanebase + d], zero16, mask=valid)
            m_sel = gt | (eq & all_eq_v)
            m_keep = eq & jnp.logical_not(all_eq_v)
            pc = plsc.cumsum(m_sel.astype(jnp.int32))
            plsc.store_scatter(su_v, [ns + pc - 1], k, mask=m_sel)
            plsc.store_scatter(si_v, [ns + pc - 1], idx, mask=m_sel)
            plsc.store_scatter(au_v, [bbase + nk], k, mask=m_keep)
            plsc.store_scatter(ai_v, [bbase + nk], idx, mask=m_keep)
            ns = ns + plsc.all_reduce_population_count(m_sel)
            nk = nk + m_keep.astype(jnp.int32)
            return ns, nk
        ns_vec, nk_vec = lax.fori_loop(0, maxc, r_body, (ns_vec, zero16))
        return ns_vec, nk_vec

    def small_round(r, k_rem_v, ns_vec, cnt_vec):
        can_dump_v = (ns_vec + _bsum(cnt_vec)) <= OUTW
        dump_s = jnp.max(jnp.where(can_dump_v, 1, 0)) == 1
        maxc = jnp.max(cnt_vec)

        def d_body(t, ns):
            pos = bbase + t
            k = plsc.load_gather(au_v, [pos])
            idx = plsc.load_gather(ai_v, [pos])
            valid = cnt_vec > t
            pc = plsc.cumsum(valid.astype(jnp.int32))
            plsc.store_scatter(su_v, [ns + pc - 1], k, mask=valid)
            plsc.store_scatter(si_v, [ns + pc - 1], idx, mask=valid)
            return ns + plsc.all_reduce_population_count(valid)
        ns_vec = lax.fori_loop(0, jnp.where(dump_s, maxc, 0), d_body, ns_vec)
        cnt_vec = jnp.where(can_dump_v, zero16, cnt_vec)
        maxc = jnp.where(dump_s, 0, maxc)
        live = maxc > 0
        n16 = jnp.where(live, jnp.int32(16), jnp.int32(0))

        def h_body(t, c):
            pos = bbase + t
            k = plsc.load_gather(au_v, [pos])
            idx = plsc.load_gather(ai_v, [pos])
            valid = cnt_vec > t
            d = _digit_of(k, idx, r)
            plsc.addupdate_scatter(h0, [lanebase + d], one16, mask=valid)
            return c
        lax.fori_loop(0, maxc, h_body, 0)

        bstar_v, k_rem2_v, all_eq_v = scan_bins(k_rem_v, r, n16)
        ns_vec, cnt_vec = route(r, bstar_v, all_eq_v, ns_vec, cnt_vec, None)
        k_rem_v = jnp.where(all_eq_v | jnp.logical_not(live), zero16, k_rem2_v)
        return k_rem_v, ns_vec, cnt_vec

    for rr in range(ROWS_PER_W):
        out_row = wid * ROWS_PER_W + rr
        in_row = (B - 1) - out_row

        if rr == 0:
            pltpu.sync_copy(y_hbm.at[in_row, SEQ - 1], row_v)
        else:
            row_dma.wait()

        if rr == 0:
            def z_body(i, c):
                for h in hists:
                    h[pl.ds(i * 16, 16)] = zero16
                return c
            lax.fori_loop(0, 256, z_body, 0)

        for q in range(4):
            su_v[pl.ds(q * 16, 16)] = min16
            si_v[pl.ds(q * 16, 16)] = neg16
        ob_v[pl.ds(48, 16)] = neg16

        def h0_body(i, c):
            bs = [plsc.bitcast(row_v[pl.ds(i * 256 + u * 16, 16)], jnp.int32)
                  for u in range(16)]
            dg = [lax.shift_right_logical(
                      b ^ (lax.shift_right_arithmetic(b, 31) | _MININT), 24)
                  + lanebase
                  for b in bs]
            for u in range(16):
                plsc.addupdate_scatter(hists[u % 4], [dg[u]], one16)
            return c
        lax.fori_loop(0, 128, h0_body, 0)

        k_rem_v = jnp.full((16,), K, jnp.int32)
        bstar_v, k_rem2_v, all_eq_v = scan_bins(k_rem_v, 0, jnp.int32(16))
        t_lo_v = (bstar_v << 24) ^ min16
        t_bits_v = jnp.where(t_lo_v >= 0, t_lo_v,
                             t_lo_v ^ jnp.int32(0x7FFFFFFF))
        t_f_v = plsc.bitcast(t_bits_v, jnp.float32)

        def b_body(i, carry):
            cnt = carry
            vs = [row_v[pl.ds(i * 256 + u * 16, 16)] for u in range(16)]
            ms = [v >= t_f_v for v in vs]
            for u in range(16):
                pos = bbase + cnt
                plsc.store_scatter(au_v, [pos],
                                   plsc.bitcast(vs[u], jnp.int32),
                                   mask=ms[u])
                plsc.store_scatter(ai_v, [pos], (i * 256 + u * 16) + lane,
                                   mask=ms[u])
                cnt = cnt + ms[u].astype(jnp.int32)
            for h in hists:
                h[pl.ds(i * 32, 16)] = zero16
                h[pl.ds(i * 32 + 16, 16)] = zero16
            return cnt
        cnt_vec = lax.fori_loop(0, 128, b_body, zero16)

        if rr + 1 < ROWS_PER_W:
            row_dma = pltpu.async_copy(
                y_hbm.at[(B - 1) - (out_row + 1), SEQ - 1], row_v, dma_sem)

        ns_vec, cnt_vec = route(0, bstar_v, all_eq_v, zero16, cnt_vec, None)
        k_rem_v = jnp.where(all_eq_v, zero16, k_rem2_v)

        for r in range(1, 6):
            k_rem_v, ns_vec, cnt_vec = small_round(r, k_rem_v, ns_vec,
                                                   cnt_vec)

        def srt_body(j, carry):
            u0, u1, u2, u3, i0, i1, i2, i3 = carry
            s = _bmax(jnp.maximum(jnp.maximum(u0, u1), jnp.maximum(u2, u3)))
            c0 = jnp.where(u0 == s, i0, -1)
            c1 = jnp.where(u1 == s, i1, -1)
            c2 = jnp.where(u2 == s, i2, -1)
            c3 = jnp.where(u3 == s, i3, -1)
            mi = _bmax(jnp.maximum(jnp.maximum(c0, c1), jnp.maximum(c2, c3)))
            plsc.store_scatter(ob_v, [jnp.full((16,), j, jnp.int32)], mi)
            outs = []
            for (u, iv) in ((u0, i0), (u1, i1), (u2, i2), (u3, i3)):
                hit = (u == s) & (iv == mi)
                outs.append(jnp.where(hit, min16, u))
                outs.append(jnp.where(hit, neg16, iv))
            return (outs[0], outs[2], outs[4], outs[6],
                    outs[1], outs[3], outs[5], outs[7])

        init = (su_v[pl.ds(0, 16)], su_v[pl.ds(16, 16)],
                su_v[pl.ds(32, 16)], su_v[pl.ds(48, 16)],
                si_v[pl.ds(0, 16)], si_v[pl.ds(16, 16)],
                si_v[pl.ds(32, 16)], si_v[pl.ds(48, 16)])
        lax.fori_loop(0, K, srt_body, init)

        pltpu.sync_copy(ob_v, out_hbm.at[out_row])


@functools.cache
def _build_topk_sc():
    return pl.kernel(
        _topk_body,
        name="topk_radix_select",
        out_type=jax.ShapeDtypeStruct((B, OUTW), jnp.int32),
        mesh=plsc.VectorSubcoreMesh(core_axis_name="c", subcore_axis_name="s",
                                    num_cores=NC, num_subcores=NS),
        scratch_types=[
            pltpu.VMEM((N,), jnp.float32),
            pltpu.VMEM((N,), jnp.int32),
            pltpu.VMEM((N,), jnp.int32),
            pltpu.VMEM((16 * 256,), jnp.int32),
            pltpu.VMEM((16 * 256,), jnp.int32),
            pltpu.VMEM((16 * 256,), jnp.int32),
            pltpu.VMEM((16 * 256,), jnp.int32),
            pltpu.VMEM((256,), jnp.int32),
            pltpu.VMEM((256,), jnp.int32),
            pltpu.VMEM((OUTW,), jnp.int32),
            pltpu.VMEM((OUTW,), jnp.int32),
            pltpu.VMEM((OUTW,), jnp.int32),
            pltpu.SemaphoreType.DMA,
        ],
        compiler_params=pltpu.CompilerParams(needs_layout_passes=False),
    )


def kernel(y_pred):
    return _build_topk_sc()(y_pred)[:, :K]
